# baseline (speedup 1.0000x reference)
import functools

import jax
import jax.numpy as jnp
from jax import lax
from jax.experimental import pallas as pl
from jax.experimental.pallas import tpu as pltpu
from jax.experimental.pallas import tpu_sc as plsc

_CHUNK = 128
_BLK = 512
_D_OUT = 256


def _sc_gather(ids8, emb_var, ef_pad, evc_pad):
    info = plsc.get_sparse_core_info()
    nc, ns = info.num_cores, info.num_subcores
    nw = nc * ns
    n_tok = ids8.shape[1]
    per_w = n_tok // nw
    n_chunks = per_w // _CHUNK
    n_groups = _CHUNK // 16

    mesh = plsc.VectorSubcoreMesh(core_axis_name="c", subcore_axis_name="s")

    @functools.partial(
        pl.kernel,
        mesh=mesh,
        compiler_params=pltpu.CompilerParams(needs_layout_passes=False),
        out_type=[
            jax.ShapeDtypeStruct((n_tok, 128), jnp.float32),
            jax.ShapeDtypeStruct((n_tok, 64), jnp.float32),
        ],
        scratch_types=[
            pltpu.VMEM((8, _CHUNK), jnp.int32),
            pltpu.VMEM((_CHUNK, 128), jnp.float32),
            pltpu.VMEM((_CHUNK, 64), jnp.float32),
            pltpu.VMEM(ef_pad.shape, jnp.float32),
            pltpu.VMEM(evc_pad.shape, jnp.float32),
            pltpu.SemaphoreType.DMA,
        ],
    )
    def k(ids_h, table_h, ef_h, evc_h, hvar_h, hsmall_h,
          idc, vrows, small, ef_v, evc_v, sem):
        wid = lax.axis_index("s") * nc + lax.axis_index("c")
        pltpu.sync_copy(ef_h, ef_v)
        pltpu.sync_copy(evc_h, evc_v)
        iota16 = lax.iota(jnp.int32, 16)

        def chunk_body(g, carry):
            base = wid * per_w + g * _CHUNK
            pltpu.sync_copy(ids_h.at[:, pl.ds(base, _CHUNK)], idc)
            h = pltpu.async_copy(table_h.at[idc.at[0]], vrows, sem)

            def group_body(gg, carry2):
                t0 = gg * 16
                toks = iota16 + t0
                vc32 = idc[1, pl.ds(t0, 16)] * 32
                f32s = [idc[2 + j, pl.ds(t0, 16)] * 32 for j in range(6)]
                for c in range(32):
                    col = jnp.full((16,), c, jnp.int32)
                    vflat = vc32 + c
                    vcv = plsc.load_gather(
                        evc_v, [vflat >> 7, vflat & 127])
                    fflat = f32s[0] + c
                    acc = plsc.load_gather(ef_v, [fflat >> 7, fflat & 127])
                    for j in range(1, 6):
                        fflat = f32s[j] + c
                        acc = acc + plsc.load_gather(
                            ef_v, [fflat >> 7, fflat & 127])
                    plsc.store_scatter(small, [toks, col], vcv)
                    plsc.store_scatter(
                        small, [toks, jnp.full((16,), c + 32, jnp.int32)],
                        acc * (1.0 / 6.0))
                return carry2

            lax.fori_loop(0, n_groups, group_body, 0)
            h.wait()
            pltpu.sync_copy(vrows, hvar_h.at[pl.ds(base, _CHUNK)])
            pltpu.sync_copy(small, hsmall_h.at[pl.ds(base, _CHUNK)])
            return carry

        lax.fori_loop(0, n_chunks, chunk_body, 0)

    return k(ids8, emb_var, ef_pad, evc_pad)


def _tc_body(hv_ref, hsm_ref, vaf_ref, w_ref, b_ref, out_ref):
    wv = w_ref[0:128, :]
    wsm = w_ref[128:192, :]
    wvaf = w_ref[192:193, :]
    acc = jnp.dot(hv_ref[...], wv, preferred_element_type=jnp.float32)
    acc = acc + jnp.dot(hsm_ref[...], wsm, preferred_element_type=jnp.float32)
    acc = acc + vaf_ref[...] * wvaf
    acc = acc + b_ref[...]
    out_ref[...] = jnp.where(acc > 0.0, acc,
                             jnp.exp(jnp.minimum(acc, 0.0)) - 1.0)


def _tc_project(hvar, hsmall, vaf, w, b2):
    n_tok = hvar.shape[0]
    grid = (n_tok // _BLK,)
    return pl.pallas_call(
        _tc_body,
        grid=grid,
        in_specs=[
            pl.BlockSpec((_BLK, 128), lambda i: (i, 0)),
            pl.BlockSpec((_BLK, 64), lambda i: (i, 0)),
            pl.BlockSpec((_BLK, 1), lambda i: (i, 0)),
            pl.BlockSpec((193, _D_OUT), lambda i: (0, 0)),
            pl.BlockSpec((1, _D_OUT), lambda i: (0, 0)),
        ],
        out_specs=pl.BlockSpec((_BLK, _D_OUT), lambda i: (i, 0)),
        out_shape=jax.ShapeDtypeStruct((n_tok, _D_OUT), jnp.float32),
        compiler_params=pltpu.CompilerParams(
            dimension_semantics=("arbitrary",)),
    )(hvar, hsmall, vaf, w, b2)


def kernel(x_omic, emb_var, emb_vc, emb_func, W, b):
    bsz, seq, _ = x_omic.shape
    n_tok = bsz * seq
    ids8 = x_omic[..., 0:8].astype(jnp.int32).reshape(n_tok, 8).T
    vaf = x_omic[..., 8].reshape(n_tok, 1)
    ef_pad = jnp.pad(emb_func, ((0, 7), (0, 0))).reshape(252, 128)
    evc_pad = jnp.pad(emb_vc, ((0, 5), (0, 0))).reshape(8, 128)
    hvar, hsmall = _sc_gather(ids8, emb_var, ef_pad, evc_pad)
    out = _tc_project(hvar, hsmall, vaf, W, b.reshape(1, -1))
    return out.reshape(bsz, seq, _D_OUT)


# speedup vs baseline: 2.0471x; 2.0471x over previous
import functools

import jax
import jax.numpy as jnp
from jax import lax
from jax.experimental import pallas as pl
from jax.experimental.pallas import tpu as pltpu
from jax.experimental.pallas import tpu_sc as plsc

_CHUNK = 128
_BLK = 1024
_D_OUT = 256


def _sc_gather(x_t, emb_var, emb_vc, emb_func):
    info = plsc.get_sparse_core_info()
    nc, ns = info.num_cores, info.num_subcores
    nw = nc * ns
    n_tok = x_t.shape[1]
    per_w = n_tok // nw
    n_chunks = per_w // _CHUNK
    n_groups = _CHUNK // 16

    mesh = plsc.VectorSubcoreMesh(core_axis_name="c", subcore_axis_name="s")

    @functools.partial(
        pl.kernel,
        mesh=mesh,
        compiler_params=pltpu.CompilerParams(
            use_tc_tiling_on_sc=False, needs_layout_passes=False),
        out_type=[
            jax.ShapeDtypeStruct((n_tok, 128), jnp.float32),
            jax.ShapeDtypeStruct((n_tok, 128), jnp.float32),
        ],
        scratch_types=[
            pltpu.VMEM((9, _CHUNK), jnp.float32),
            pltpu.VMEM((_CHUNK,), jnp.int32),
            pltpu.VMEM((_CHUNK,), jnp.int32),
            pltpu.VMEM((6, _CHUNK), jnp.int32),
            pltpu.VMEM((_CHUNK, 128), jnp.float32),
            pltpu.VMEM((_CHUNK, 32), jnp.float32),
            pltpu.VMEM((6, _CHUNK, 32), jnp.float32),
            pltpu.VMEM((_CHUNK, 128), jnp.float32),
            pltpu.SemaphoreType.DMA,
        ],
    )
    def k(xt_h, table_h, vc_tab_h, func_tab_h, hvar_h, hsmall_h,
          xch, vidx, vcidx, fidx, vrows, vcrows, frows, small, sem):
        wid = lax.axis_index("s") * nc + lax.axis_index("c")
        iota16 = lax.iota(jnp.int32, 16)
        zero16 = jnp.zeros((16,), jnp.float32)
        col64 = jnp.full((16,), 64, jnp.int32)

        def zero_body(t, carry):
            for h in range(4):
                small[t, pl.ds(64 + h * 16, 16)] = zero16
            return carry

        lax.fori_loop(0, _CHUNK, zero_body, 0)

        def chunk_body(g, carry):
            base = wid * per_w + g * _CHUNK
            pltpu.sync_copy(xt_h.at[:, pl.ds(base, _CHUNK)], xch)

            def idx_body(gg, carry2):
                sl = pl.ds(gg * 16, 16)
                vidx[sl] = xch[0, sl].astype(jnp.int32)
                vcidx[sl] = xch[1, sl].astype(jnp.int32)
                for j in range(6):
                    fidx[j, sl] = xch[2 + j, sl].astype(jnp.int32)
                return carry2

            lax.fori_loop(0, n_groups, idx_body, 0)

            copies = [
                pltpu.async_copy(table_h.at[vidx], vrows, sem),
                pltpu.async_copy(vc_tab_h.at[vcidx], vcrows, sem),
            ]
            for j in range(6):
                copies.append(
                    pltpu.async_copy(func_tab_h.at[fidx.at[j]], frows.at[j], sem))
            for c in copies:
                c.wait()

            def pool_body(t, carry2):
                for h in range(2):
                    sl = pl.ds(h * 16, 16)
                    small[t, sl] = vcrows[t, sl]
                    acc = frows[0, t, sl]
                    for j in range(1, 6):
                        acc = acc + frows[j, t, sl]
                    small[t, pl.ds(32 + h * 16, 16)] = acc * (1.0 / 6.0)
                return carry2

            lax.fori_loop(0, _CHUNK, pool_body, 0)

            def vaf_body(gg, carry2):
                t0 = gg * 16
                vafv = xch[8, pl.ds(t0, 16)]
                plsc.store_scatter(small, [iota16 + t0, col64], vafv)
                return carry2

            lax.fori_loop(0, n_groups, vaf_body, 0)

            pltpu.sync_copy(vrows, hvar_h.at[pl.ds(base, _CHUNK)])
            pltpu.sync_copy(small, hsmall_h.at[pl.ds(base, _CHUNK)])
            return carry

        lax.fori_loop(0, n_chunks, chunk_body, 0)

    return k(x_t, emb_var, emb_vc, emb_func)


def _tc_body(hv_ref, hsm_ref, w_ref, wsm_ref, b_ref, out_ref):
    wv = w_ref[0:128, :]
    acc = jnp.dot(hv_ref[...], wv, preferred_element_type=jnp.float32)
    acc = acc + jnp.dot(hsm_ref[...], wsm_ref[...],
                        preferred_element_type=jnp.float32)
    acc = acc + b_ref[...]
    out_ref[...] = jnp.where(acc > 0.0, acc,
                             jnp.exp(jnp.minimum(acc, 0.0)) - 1.0)


def _tc_project(hvar, hsmall, w, wsm_pad, b2):
    n_tok = hvar.shape[0]
    grid = (n_tok // _BLK,)
    return pl.pallas_call(
        _tc_body,
        grid=grid,
        in_specs=[
            pl.BlockSpec((_BLK, 128), lambda i: (i, 0)),
            pl.BlockSpec((_BLK, 128), lambda i: (i, 0)),
            pl.BlockSpec((193, _D_OUT), lambda i: (0, 0)),
            pl.BlockSpec((128, _D_OUT), lambda i: (0, 0)),
            pl.BlockSpec((1, _D_OUT), lambda i: (0, 0)),
        ],
        out_specs=pl.BlockSpec((_BLK, _D_OUT), lambda i: (i, 0)),
        out_shape=jax.ShapeDtypeStruct((n_tok, _D_OUT), jnp.float32),
        compiler_params=pltpu.CompilerParams(
            dimension_semantics=("parallel",)),
    )(hvar, hsmall, w, wsm_pad, b2)


def kernel(x_omic, emb_var, emb_vc, emb_func, W, b):
    bsz, seq, _ = x_omic.shape
    n_tok = bsz * seq
    x_t = x_omic.transpose(2, 1, 0).reshape(9, n_tok)
    wsm_pad = jnp.pad(W[128:193], ((0, 63), (0, 0)))
    hvar, hsmall = _sc_gather(x_t, emb_var, emb_vc, emb_func)
    out2d = _tc_project(hvar, hsmall, W, wsm_pad, b.reshape(1, -1))
    return out2d.reshape(seq, bsz, _D_OUT).transpose(1, 0, 2)
